# Initial kernel scaffold; baseline (speedup 1.0000x reference)
import jax
import jax.numpy as jnp
from jax.experimental import pallas as pl


def kernel(x, edge_index, batch, device, W0, b0, W1, b1, W2, b2, bn0_g, bn0_b, bn1_g, bn1_b, bn2_g, bn2_b, bnfc_g, bnfc_b, Wfc, bfc, bnh_g, bnh_b, Wcls, bcls):
    raise NotImplementedError("write your pallas kernel here")


# speedup vs baseline: 16.6587x; 16.6587x over previous
import functools

import jax
import jax.numpy as jnp
from jax import lax
from jax.experimental import pallas as pl
from jax.experimental.pallas import tpu as pltpu
from jax.experimental.pallas import tpu_sc as plsc

N = 10000
E = 320000
F = 128
H = 128
C = 10
G = 128

NC = 2
NS = 16
NW = NC * NS
EPT = E // NW
K = 125
NCHUNK = EPT // K
ROWS_PT = 632
N_PAD = NS * ROWS_PT
ZR = 8


def _sc_mesh():
    return plsc.VectorSubcoreMesh(core_axis_name="c", subcore_axis_name="s")


def _deg_body(dst_hbm, out_hbm, dst_v, ones_v, zbuf, acc):
    c = lax.axis_index("c")
    s = lax.axis_index("s")
    w = c * NS + s

    for r in range(ZR):
        for k in range(8):
            zbuf[r, pl.ds(k * 16, 16)] = jnp.zeros((16,), jnp.float32)

    for r in range(K):
        for k in range(8):
            ones_v[r, pl.ds(k * 16, 16)] = jnp.ones((16,), jnp.float32)

    def zcopy(i, _):
        pltpu.sync_copy(zbuf, acc.at[pl.ds(s * ROWS_PT + i * ZR, ZR)])
        return _
    lax.fori_loop(0, ROWS_PT // ZR, zcopy, None)
    plsc.subcore_barrier()

    pltpu.sync_copy(dst_hbm.at[w], dst_v)

    def chunk(j, _):
        pltpu.sync_copy(ones_v, acc.at[dst_v.at[j]], add=True)
        return _
    lax.fori_loop(0, NCHUNK, chunk, None)
    plsc.subcore_barrier()

    pltpu.sync_copy(acc.at[pl.ds(s * ROWS_PT, ROWS_PT)],
                    out_hbm.at[c, pl.ds(s * ROWS_PT, ROWS_PT)])


def _deg_kernel(dst_r):
    return pl.kernel(
        _deg_body,
        out_type=jax.ShapeDtypeStruct((NC, N_PAD, H), jnp.float32),
        mesh=_sc_mesh(),
        scratch_types=[
            pltpu.VMEM((NCHUNK, K), jnp.int32),
            pltpu.VMEM((K, H), jnp.float32),
            pltpu.VMEM((ZR, H), jnp.float32),
            pltpu.VMEM_SHARED((N_PAD, H), jnp.float32),
        ],
    )(dst_r)


def _scatter_body(m2_hbm, src_hbm, dst_hbm, out_hbm,
                  src_v, dst_v, rows_v, zbuf, acc, sem):
    c = lax.axis_index("c")
    s = lax.axis_index("s")
    w = c * NS + s

    for r in range(ZR):
        for k in range(8):
            zbuf[r, pl.ds(k * 16, 16)] = jnp.zeros((16,), jnp.float32)

    def zcopy(i, _):
        pltpu.sync_copy(zbuf, acc.at[pl.ds(s * ROWS_PT + i * ZR, ZR)])
        return _
    lax.fori_loop(0, ROWS_PT // ZR, zcopy, None)
    plsc.subcore_barrier()

    pltpu.sync_copy(src_hbm.at[w], src_v)
    pltpu.sync_copy(dst_hbm.at[w], dst_v)

    def chunk(j, _):
        pltpu.async_copy(m2_hbm.at[src_v.at[j]], rows_v, sem).wait()
        pltpu.sync_copy(rows_v, acc.at[dst_v.at[j]], add=True)
        return _
    lax.fori_loop(0, NCHUNK, chunk, None)
    plsc.subcore_barrier()

    pltpu.sync_copy(acc.at[pl.ds(s * ROWS_PT, ROWS_PT)],
                    out_hbm.at[c, pl.ds(s * ROWS_PT, ROWS_PT)])


def _scatter_kernel(m2, src_r, dst_r):
    return pl.kernel(
        _scatter_body,
        out_type=jax.ShapeDtypeStruct((NC, N_PAD, H), jnp.float32),
        mesh=_sc_mesh(),
        scratch_types=[
            pltpu.VMEM((NCHUNK, K), jnp.int32),
            pltpu.VMEM((NCHUNK, K), jnp.int32),
            pltpu.VMEM((K, H), jnp.float32),
            pltpu.VMEM((ZR, H), jnp.float32),
            pltpu.VMEM_SHARED((N_PAD, H), jnp.float32),
            pltpu.SemaphoreType.DMA,
        ],
    )(m2, src_r, dst_r)


def _bn(h, g, b):
    mu = jnp.mean(h, axis=0, keepdims=True)
    var = jnp.mean((h - mu) ** 2, axis=0, keepdims=True)
    return (h - mu) * jax.lax.rsqrt(var + 1e-5) * g + b


def _tc_pro_body(x_ref, degp_ref, bng_ref, bnb_ref, w_ref,
                 dis_ref, m2_ref):
    dp = degp_ref[...]
    deg = dp[0, :N, 0:1] + dp[1, :N, 0:1] + 1.0
    dis = jax.lax.rsqrt(deg)
    dis_ref[...] = dis
    h_ = _bn(x_ref[...], bng_ref[...], bnb_ref[...])
    m2_ref[...] = dis * jnp.dot(h_, w_ref[...],
                                preferred_element_type=jnp.float32)


def _tc_mid_body(h_ref, m2_ref, parts_ref, dis_ref, b_ref,
                 bng_ref, bnb_ref, w_ref, hn_ref, m2n_ref):
    dis = dis_ref[...]
    pr = parts_ref[...]
    agg = pr[0, :N] + pr[1, :N] + m2_ref[...]
    out = dis * agg + b_ref[...]
    h = h_ref[...] + jnp.maximum(out, 0.0)
    hn_ref[...] = h
    h_ = _bn(h, bng_ref[...], bnb_ref[...])
    m2n_ref[...] = dis * jnp.dot(h_, w_ref[...],
                                 preferred_element_type=jnp.float32)


def _tc_epi_body(h_ref, m2_ref, parts_ref, dis_ref, b_ref, batch_ref,
                 bnfcg_ref, bnfcb_ref, wfc_ref, bfc_ref,
                 bnhg_ref, bnhb_ref, wcls_ref, bcls_ref, out_ref):
    dis = dis_ref[...]
    pr = parts_ref[...]
    agg = pr[0, :N] + pr[1, :N] + m2_ref[...]
    h = h_ref[...] + jnp.maximum(dis * agg + b_ref[...], 0.0)

    gid = jax.lax.broadcasted_iota(jnp.int32, (N, G), 1)
    p = (batch_ref[...] == gid).astype(jnp.float32)
    g = jax.lax.dot_general(p, h, (((0,), (0,)), ((), ())),
                            preferred_element_type=jnp.float32)

    g = _bn(g, bnfcg_ref[...], bnfcb_ref[...])
    g = jnp.maximum(jnp.dot(g, wfc_ref[...],
                            preferred_element_type=jnp.float32) + bfc_ref[...],
                    0.0)
    g = _bn(g, bnhg_ref[...], bnhb_ref[...])
    logits = jnp.dot(g, wcls_ref[...],
                     preferred_element_type=jnp.float32) + bcls_ref[...]
    m = jnp.max(logits, axis=-1, keepdims=True)
    lse = m + jnp.log(jnp.sum(jnp.exp(logits - m), axis=-1, keepdims=True))
    out_ref[...] = logits - lse


def _tc_call(body, out_shapes, *args):
    return pl.pallas_call(
        body,
        out_shape=out_shapes,
    )(*args)


def kernel(x, edge_index, batch, device, W0, b0, W1, b1, W2, b2,
           bn0_g, bn0_b, bn1_g, bn1_b, bn2_g, bn2_b,
           bnfc_g, bnfc_b, Wfc, bfc, bnh_g, bnh_b, Wcls, bcls):
    src_r = edge_index[0].astype(jnp.int32).reshape(NW, NCHUNK, K)
    dst_r = edge_index[1].astype(jnp.int32).reshape(NW, NCHUNK, K)
    batch2 = batch.astype(jnp.int32).reshape(N, 1)

    degp = _deg_kernel(dst_r)
    dis, m2 = _tc_call(
        _tc_pro_body,
        [jax.ShapeDtypeStruct((N, 1), jnp.float32),
         jax.ShapeDtypeStruct((N, H), jnp.float32)],
        x, degp, bn0_g.reshape(1, F), bn0_b.reshape(1, F), W0)

    h = x
    for bb, bg, bnb, W in ((b0, bn1_g, bn1_b, W1), (b1, bn2_g, bn2_b, W2)):
        parts = _scatter_kernel(m2, src_r, dst_r)
        h, m2 = _tc_call(
            _tc_mid_body,
            [jax.ShapeDtypeStruct((N, H), jnp.float32),
             jax.ShapeDtypeStruct((N, H), jnp.float32)],
            h, m2, parts, dis, bb.reshape(1, H),
            bg.reshape(1, H), bnb.reshape(1, H), W)

    parts = _scatter_kernel(m2, src_r, dst_r)
    out = _tc_call(
        _tc_epi_body,
        jax.ShapeDtypeStruct((G, C), jnp.float32),
        h, m2, parts, dis, b2.reshape(1, H), batch2,
        bnfc_g.reshape(1, H), bnfc_b.reshape(1, H), Wfc, bfc.reshape(1, H),
        bnh_g.reshape(1, H), bnh_b.reshape(1, H), Wcls, bcls.reshape(1, C))
    return out


# speedup vs baseline: 21.3888x; 1.2839x over previous
import functools

import jax
import jax.numpy as jnp
from jax import lax
from jax.experimental import pallas as pl
from jax.experimental.pallas import tpu as pltpu
from jax.experimental.pallas import tpu_sc as plsc

N = 10000
E = 320000
F = 128
H = 128
C = 10
G = 128

NC = 2
NS = 16
NW = NC * NS
EPT = E // NW
K = 125
NCHUNK = EPT // K
ROWS_PT = 632
N_PAD = NS * ROWS_PT
ZR = 8
GRP = 16


def _sc_mesh():
    return plsc.VectorSubcoreMesh(core_axis_name="c", subcore_axis_name="s")


def _deg_body(dst_hbm, out_hbm, dst_v, ones_v, zbuf, acc):
    c = lax.axis_index("c")
    s = lax.axis_index("s")
    w = c * NS + s

    for r in range(ZR):
        for k in range(8):
            zbuf[r, pl.ds(k * 16, 16)] = jnp.zeros((16,), jnp.float32)

    for r in range(K):
        for k in range(8):
            ones_v[r, pl.ds(k * 16, 16)] = jnp.ones((16,), jnp.float32)

    def zcopy(i, _):
        pltpu.sync_copy(zbuf, acc.at[pl.ds(s * ROWS_PT + i * ZR, ZR)])
        return _
    lax.fori_loop(0, ROWS_PT // ZR, zcopy, None)
    plsc.subcore_barrier()

    pltpu.sync_copy(dst_hbm.at[w], dst_v)

    def chunk(j, _):
        pltpu.sync_copy(ones_v, acc.at[dst_v.at[j]], add=True)
        return _
    lax.fori_loop(0, NCHUNK, chunk, None)
    plsc.subcore_barrier()

    pltpu.sync_copy(acc.at[pl.ds(s * ROWS_PT, ROWS_PT)],
                    out_hbm.at[c, pl.ds(s * ROWS_PT, ROWS_PT)])


def _deg_kernel(dst_r):
    return pl.kernel(
        _deg_body,
        out_type=jax.ShapeDtypeStruct((NC, N_PAD, H), jnp.float32),
        mesh=_sc_mesh(),
        scratch_types=[
            pltpu.VMEM((NCHUNK, K), jnp.int32),
            pltpu.VMEM((K, H), jnp.float32),
            pltpu.VMEM((ZR, H), jnp.float32),
            pltpu.VMEM_SHARED((N_PAD, H), jnp.float32),
        ],
    )(dst_r)


def _scatter_body(m2_hbm, src_hbm, dst_hbm, out_hbm,
                  src_v, dst_v, rows0, rows1, zbuf, acc, sem0, sem1):
    c = lax.axis_index("c")
    s = lax.axis_index("s")
    w = c * NS + s

    for r in range(ZR):
        for k in range(8):
            zbuf[r, pl.ds(k * 16, 16)] = jnp.zeros((16,), jnp.float32)

    def zcopy(i, _):
        pltpu.sync_copy(zbuf, acc.at[pl.ds(s * ROWS_PT + i * ZR, ZR)])
        return _
    lax.fori_loop(0, ROWS_PT // ZR, zcopy, None)
    plsc.subcore_barrier()

    def group(g, _):
        pltpu.sync_copy(src_hbm.at[w, pl.ds(g * GRP, GRP)], src_v)
        pltpu.sync_copy(dst_hbm.at[w, pl.ds(g * GRP, GRP)], dst_v)
        pltpu.async_copy(m2_hbm.at[src_v.at[0]], rows0, sem0)
        pltpu.async_copy(m2_hbm.at[src_v.at[1]], rows1, sem1)

        def step(i, _2):
            l0 = 2 * i
            l1 = 2 * i + 1
            pltpu.make_async_copy(m2_hbm.at[src_v.at[l0]], rows0, sem0).wait()
            pltpu.sync_copy(rows0, acc.at[dst_v.at[l0]], add=True)
            pltpu.async_copy(m2_hbm.at[src_v.at[(l0 + 2) % GRP]], rows0, sem0)
            pltpu.make_async_copy(m2_hbm.at[src_v.at[l1]], rows1, sem1).wait()
            pltpu.sync_copy(rows1, acc.at[dst_v.at[l1]], add=True)
            pltpu.async_copy(m2_hbm.at[src_v.at[(l1 + 2) % GRP]], rows1, sem1)
            return _2
        lax.fori_loop(0, GRP // 2, step, None)
        pltpu.make_async_copy(m2_hbm.at[src_v.at[0]], rows0, sem0).wait()
        pltpu.make_async_copy(m2_hbm.at[src_v.at[1]], rows1, sem1).wait()
        return _
    lax.fori_loop(0, NCHUNK // GRP, group, None)
    plsc.subcore_barrier()

    pltpu.sync_copy(acc.at[pl.ds(s * ROWS_PT, ROWS_PT)],
                    out_hbm.at[c, pl.ds(s * ROWS_PT, ROWS_PT)])


def _scatter_kernel(m2, src_r, dst_r):
    return pl.kernel(
        _scatter_body,
        out_type=jax.ShapeDtypeStruct((NC, N_PAD, H), jnp.float32),
        mesh=_sc_mesh(),
        scratch_types=[
            pltpu.VMEM((GRP, K), jnp.int32),
            pltpu.VMEM((GRP, K), jnp.int32),
            pltpu.VMEM((K, H), jnp.float32),
            pltpu.VMEM((K, H), jnp.float32),
            pltpu.VMEM((ZR, H), jnp.float32),
            pltpu.VMEM_SHARED((N_PAD, H), jnp.float32),
            pltpu.SemaphoreType.DMA,
            pltpu.SemaphoreType.DMA,
        ],
    )(m2, src_r, dst_r)


def _bn(h, g, b):
    mu = jnp.mean(h, axis=0, keepdims=True)
    var = jnp.mean((h - mu) ** 2, axis=0, keepdims=True)
    return (h - mu) * jax.lax.rsqrt(var + 1e-5) * g + b


def _tc_pro_body(x_ref, degp_ref, bng_ref, bnb_ref, w_ref,
                 dis_ref, m2_ref):
    dp = degp_ref[...]
    deg = dp[0, :N, 0:1] + dp[1, :N, 0:1] + 1.0
    dis = jax.lax.rsqrt(deg)
    dis_ref[...] = dis
    h_ = _bn(x_ref[...], bng_ref[...], bnb_ref[...])
    m2_ref[...] = dis * jnp.dot(h_, w_ref[...],
                                preferred_element_type=jnp.float32)


def _tc_mid_body(h_ref, m2_ref, parts_ref, dis_ref, b_ref,
                 bng_ref, bnb_ref, w_ref, hn_ref, m2n_ref):
    dis = dis_ref[...]
    pr = parts_ref[...]
    agg = pr[0, :N] + pr[1, :N] + m2_ref[...]
    out = dis * agg + b_ref[...]
    h = h_ref[...] + jnp.maximum(out, 0.0)
    hn_ref[...] = h
    h_ = _bn(h, bng_ref[...], bnb_ref[...])
    m2n_ref[...] = dis * jnp.dot(h_, w_ref[...],
                                 preferred_element_type=jnp.float32)


def _tc_epi_body(h_ref, m2_ref, parts_ref, dis_ref, b_ref, batch_ref,
                 bnfcg_ref, bnfcb_ref, wfc_ref, bfc_ref,
                 bnhg_ref, bnhb_ref, wcls_ref, bcls_ref, out_ref):
    dis = dis_ref[...]
    pr = parts_ref[...]
    agg = pr[0, :N] + pr[1, :N] + m2_ref[...]
    h = h_ref[...] + jnp.maximum(dis * agg + b_ref[...], 0.0)

    gid = jax.lax.broadcasted_iota(jnp.int32, (N, G), 1)
    p = (batch_ref[...] == gid).astype(jnp.float32)
    g = jax.lax.dot_general(p, h, (((0,), (0,)), ((), ())),
                            preferred_element_type=jnp.float32)

    g = _bn(g, bnfcg_ref[...], bnfcb_ref[...])
    g = jnp.maximum(jnp.dot(g, wfc_ref[...],
                            preferred_element_type=jnp.float32) + bfc_ref[...],
                    0.0)
    g = _bn(g, bnhg_ref[...], bnhb_ref[...])
    logits = jnp.dot(g, wcls_ref[...],
                     preferred_element_type=jnp.float32) + bcls_ref[...]
    m = jnp.max(logits, axis=-1, keepdims=True)
    lse = m + jnp.log(jnp.sum(jnp.exp(logits - m), axis=-1, keepdims=True))
    out_ref[...] = logits - lse


def _tc_call(body, out_shapes, *args):
    return pl.pallas_call(
        body,
        out_shape=out_shapes,
    )(*args)


def kernel(x, edge_index, batch, device, W0, b0, W1, b1, W2, b2,
           bn0_g, bn0_b, bn1_g, bn1_b, bn2_g, bn2_b,
           bnfc_g, bnfc_b, Wfc, bfc, bnh_g, bnh_b, Wcls, bcls):
    src_r = edge_index[0].astype(jnp.int32).reshape(NW, NCHUNK, K)
    dst_r = edge_index[1].astype(jnp.int32).reshape(NW, NCHUNK, K)
    batch2 = batch.astype(jnp.int32).reshape(N, 1)

    degp = _deg_kernel(dst_r)
    dis, m2 = _tc_call(
        _tc_pro_body,
        [jax.ShapeDtypeStruct((N, 1), jnp.float32),
         jax.ShapeDtypeStruct((N, H), jnp.float32)],
        x, degp, bn0_g.reshape(1, F), bn0_b.reshape(1, F), W0)

    h = x
    for bb, bg, bnb, W in ((b0, bn1_g, bn1_b, W1), (b1, bn2_g, bn2_b, W2)):
        parts = _scatter_kernel(m2, src_r, dst_r)
        h, m2 = _tc_call(
            _tc_mid_body,
            [jax.ShapeDtypeStruct((N, H), jnp.float32),
             jax.ShapeDtypeStruct((N, H), jnp.float32)],
            h, m2, parts, dis, bb.reshape(1, H),
            bg.reshape(1, H), bnb.reshape(1, H), W)

    parts = _scatter_kernel(m2, src_r, dst_r)
    out = _tc_call(
        _tc_epi_body,
        jax.ShapeDtypeStruct((G, C), jnp.float32),
        h, m2, parts, dis, b2.reshape(1, H), batch2,
        bnfc_g.reshape(1, H), bnfc_b.reshape(1, H), Wfc, bfc.reshape(1, H),
        bnh_g.reshape(1, H), bnh_b.reshape(1, H), Wcls, bcls.reshape(1, C))
    return out


# speedup vs baseline: 22.6365x; 1.0583x over previous
import functools

import jax
import jax.numpy as jnp
from jax import lax
from jax.experimental import pallas as pl
from jax.experimental.pallas import tpu as pltpu
from jax.experimental.pallas import tpu_sc as plsc

N = 10000
E = 320000
F = 128
H = 128
C = 10
G = 128

NC = 2
NS = 16
NW = NC * NS
EPT = E // NW
K = 80
NCHUNK = EPT // K
ROWS_PT = 632
N_PAD = NS * ROWS_PT
ZR = 8
GRP = 25
NGRP = NCHUNK // GRP


def _sc_mesh():
    return plsc.VectorSubcoreMesh(core_axis_name="c", subcore_axis_name="s")


def _deg_body(dst_hbm, out_hbm, dst_v, ones_v, zbuf, acc, sem):
    c = lax.axis_index("c")
    s = lax.axis_index("s")
    w = c * NS + s

    for r in range(ZR):
        for k in range(8):
            zbuf[r, pl.ds(k * 16, 16)] = jnp.zeros((16,), jnp.float32)

    for r in range(K):
        for k in range(8):
            ones_v[r, pl.ds(k * 16, 16)] = jnp.ones((16,), jnp.float32)

    def zcopy(i, _):
        pltpu.sync_copy(zbuf, acc.at[pl.ds(s * ROWS_PT + i * ZR, ZR)])
        return _
    lax.fori_loop(0, ROWS_PT // ZR, zcopy, None)
    plsc.subcore_barrier()

    pltpu.sync_copy(dst_hbm.at[w], dst_v)

    def dgrp(g, _):
        def issue(l, _2):
            pltpu.async_copy(ones_v, acc.at[dst_v.at[g, l]], sem, add=True)
            return _2
        lax.fori_loop(0, GRP, issue, None)

        def drain(l, _2):
            pltpu.make_async_copy(ones_v, acc.at[dst_v.at[g, 0]], sem).wait()
            return _2
        lax.fori_loop(0, GRP, drain, None)
        return _
    lax.fori_loop(0, NGRP, dgrp, None)
    plsc.subcore_barrier()

    pltpu.sync_copy(acc.at[pl.ds(s * ROWS_PT, ROWS_PT)],
                    out_hbm.at[c, pl.ds(s * ROWS_PT, ROWS_PT)])


def _deg_kernel(dst_r):
    return pl.kernel(
        _deg_body,
        out_type=jax.ShapeDtypeStruct((NC, N_PAD, H), jnp.float32),
        mesh=_sc_mesh(),
        scratch_types=[
            pltpu.VMEM((NGRP, GRP, K), jnp.int32),
            pltpu.VMEM((K, H), jnp.float32),
            pltpu.VMEM((ZR, H), jnp.float32),
            pltpu.VMEM_SHARED((N_PAD, H), jnp.float32),
            pltpu.SemaphoreType.DMA,
        ],
    )(dst_r)


def _scatter_kernel(m2, src_r, dst_r):
    def body(m2_hbm, src_hbm, dst_hbm, out_hbm,
             src_v, dst_v, rows0, rows1, rows2, zbuf, acc,
             gs0, gs1, gs2, ss0, ss1, ss2):
        c = lax.axis_index("c")
        s = lax.axis_index("s")
        w = c * NS + s
        rows = (rows0, rows1, rows2)
        gs = (gs0, gs1, gs2)
        ss = (ss0, ss1, ss2)

        for r in range(ZR):
            for k in range(8):
                zbuf[r, pl.ds(k * 16, 16)] = jnp.zeros((16,), jnp.float32)

        def zcopy(i, _):
            pltpu.sync_copy(zbuf, acc.at[pl.ds(s * ROWS_PT + i * ZR, ZR)])
            return _
        lax.fori_loop(0, ROWS_PT // ZR, zcopy, None)
        plsc.subcore_barrier()

        def gath(l, b):
            pltpu.async_copy(m2_hbm.at[src_v.at[l]], rows[b], gs[b])

        def gath_wait(b):
            pltpu.make_async_copy(m2_hbm.at[src_v.at[0]], rows[b],
                                  gs[b]).wait()

        def scat(l, b):
            pltpu.async_copy(rows[b], acc.at[dst_v.at[l]], ss[b], add=True)

        def scat_wait(b):
            pltpu.make_async_copy(rows[b], acc.at[dst_v.at[0]],
                                  ss[b]).wait()

        def group(g, _):
            pltpu.sync_copy(src_hbm.at[w, g], src_v)
            pltpu.sync_copy(dst_hbm.at[w, g], dst_v)
            gath(0, 0)
            gath(1, 1)
            gath(2, 2)
            gath_wait(0)
            scat(0, 0)
            gath_wait(1)
            scat(1, 1)

            def steps(i, _2):
                base = 3 * i + 2
                for d in range(3):
                    l = base + d
                    scat_wait(d)
                    gath(l + 1, d)
                    gath_wait((2 + d) % 3)
                    scat(l, (2 + d) % 3)
                return _2
            lax.fori_loop(0, (GRP - 4) // 3, steps, None)
            scat_wait(0)
            gath(GRP - 1, 0)
            gath_wait((GRP - 2) % 3)
            scat(GRP - 2, (GRP - 2) % 3)
            scat_wait(1)
            gath_wait(0)
            scat(GRP - 1, 0)
            scat_wait(2)
            scat_wait(0)
            return _
        lax.fori_loop(0, NGRP, group, None)
        plsc.subcore_barrier()

        pltpu.sync_copy(acc.at[pl.ds(s * ROWS_PT, ROWS_PT)],
                        out_hbm.at[c, pl.ds(s * ROWS_PT, ROWS_PT)])

    return pl.kernel(
        body,
        out_type=jax.ShapeDtypeStruct((NC, N_PAD, H), jnp.float32),
        mesh=_sc_mesh(),
        scratch_types=[
            pltpu.VMEM((GRP, K), jnp.int32),
            pltpu.VMEM((GRP, K), jnp.int32),
            pltpu.VMEM((K, H), jnp.float32),
            pltpu.VMEM((K, H), jnp.float32),
            pltpu.VMEM((K, H), jnp.float32),
            pltpu.VMEM((ZR, H), jnp.float32),
            pltpu.VMEM_SHARED((N_PAD, H), jnp.float32),
            pltpu.SemaphoreType.DMA,
            pltpu.SemaphoreType.DMA,
            pltpu.SemaphoreType.DMA,
            pltpu.SemaphoreType.DMA,
            pltpu.SemaphoreType.DMA,
            pltpu.SemaphoreType.DMA,
        ],
    )(m2, src_r, dst_r)


def _bn(h, g, b):
    mu = jnp.mean(h, axis=0, keepdims=True)
    var = jnp.mean((h - mu) ** 2, axis=0, keepdims=True)
    return (h - mu) * jax.lax.rsqrt(var + 1e-5) * g + b


def _tc_pro_body(x_ref, degp_ref, bng_ref, bnb_ref, w_ref,
                 dis_ref, m2_ref):
    dp = degp_ref[...]
    deg = dp[0, :N, 0:1] + dp[1, :N, 0:1] + 1.0
    dis = jax.lax.rsqrt(deg)
    dis_ref[...] = dis
    h_ = _bn(x_ref[...], bng_ref[...], bnb_ref[...])
    m2_ref[...] = dis * jnp.dot(h_, w_ref[...],
                                preferred_element_type=jnp.float32)


def _tc_mid_body(h_ref, m2_ref, parts_ref, dis_ref, b_ref,
                 bng_ref, bnb_ref, w_ref, hn_ref, m2n_ref):
    dis = dis_ref[...]
    pr = parts_ref[...]
    agg = pr[0, :N] + pr[1, :N] + m2_ref[...]
    out = dis * agg + b_ref[...]
    h = h_ref[...] + jnp.maximum(out, 0.0)
    hn_ref[...] = h
    h_ = _bn(h, bng_ref[...], bnb_ref[...])
    m2n_ref[...] = dis * jnp.dot(h_, w_ref[...],
                                 preferred_element_type=jnp.float32)


def _tc_epi_body(h_ref, m2_ref, parts_ref, dis_ref, b_ref, batch_ref,
                 bnfcg_ref, bnfcb_ref, wfc_ref, bfc_ref,
                 bnhg_ref, bnhb_ref, wcls_ref, bcls_ref, out_ref):
    dis = dis_ref[...]
    pr = parts_ref[...]
    agg = pr[0, :N] + pr[1, :N] + m2_ref[...]
    h = h_ref[...] + jnp.maximum(dis * agg + b_ref[...], 0.0)

    gid = jax.lax.broadcasted_iota(jnp.int32, (N, G), 1)
    p = (batch_ref[...] == gid).astype(jnp.float32)
    g = jax.lax.dot_general(p, h, (((0,), (0,)), ((), ())),
                            preferred_element_type=jnp.float32)

    g = _bn(g, bnfcg_ref[...], bnfcb_ref[...])
    g = jnp.maximum(jnp.dot(g, wfc_ref[...],
                            preferred_element_type=jnp.float32) + bfc_ref[...],
                    0.0)
    g = _bn(g, bnhg_ref[...], bnhb_ref[...])
    logits = jnp.dot(g, wcls_ref[...],
                     preferred_element_type=jnp.float32) + bcls_ref[...]
    m = jnp.max(logits, axis=-1, keepdims=True)
    lse = m + jnp.log(jnp.sum(jnp.exp(logits - m), axis=-1, keepdims=True))
    out_ref[...] = logits - lse


def _tc_call(body, out_shapes, *args):
    return pl.pallas_call(
        body,
        out_shape=out_shapes,
    )(*args)


def kernel(x, edge_index, batch, device, W0, b0, W1, b1, W2, b2,
           bn0_g, bn0_b, bn1_g, bn1_b, bn2_g, bn2_b,
           bnfc_g, bnfc_b, Wfc, bfc, bnh_g, bnh_b, Wcls, bcls):
    src_r = edge_index[0].astype(jnp.int32).reshape(NW, NGRP, GRP, K)
    dst_r = edge_index[1].astype(jnp.int32).reshape(NW, NGRP, GRP, K)
    batch2 = batch.astype(jnp.int32).reshape(N, 1)

    degp = _deg_kernel(dst_r)
    dis, m2 = _tc_call(
        _tc_pro_body,
        [jax.ShapeDtypeStruct((N, 1), jnp.float32),
         jax.ShapeDtypeStruct((N, H), jnp.float32)],
        x, degp, bn0_g.reshape(1, F), bn0_b.reshape(1, F), W0)

    h = x
    for bb, bg, bnb, W in ((b0, bn1_g, bn1_b, W1), (b1, bn2_g, bn2_b, W2)):
        parts = _scatter_kernel(m2, src_r, dst_r)
        h, m2 = _tc_call(
            _tc_mid_body,
            [jax.ShapeDtypeStruct((N, H), jnp.float32),
             jax.ShapeDtypeStruct((N, H), jnp.float32)],
            h, m2, parts, dis, bb.reshape(1, H),
            bg.reshape(1, H), bnb.reshape(1, H), W)

    parts = _scatter_kernel(m2, src_r, dst_r)
    out = _tc_call(
        _tc_epi_body,
        jax.ShapeDtypeStruct((G, C), jnp.float32),
        h, m2, parts, dis, b2.reshape(1, H), batch2,
        bnfc_g.reshape(1, H), bnfc_b.reshape(1, H), Wfc, bfc.reshape(1, H),
        bnh_g.reshape(1, H), bnh_b.reshape(1, H), Wcls, bcls.reshape(1, C))
    return out


# speedup vs baseline: 22.6795x; 1.0019x over previous
import functools

import jax
import jax.numpy as jnp
from jax import lax
from jax.experimental import pallas as pl
from jax.experimental.pallas import tpu as pltpu
from jax.experimental.pallas import tpu_sc as plsc

N = 10000
E = 320000
F = 128
H = 128
C = 10
G = 128

NC = 2
NS = 16
NW = NC * NS
EPT = E // NW
K = 50
NCHUNK = EPT // K
ROWS_PT = 632
N_PAD = NS * ROWS_PT
ZR = 8
GRP = 40
NGRP = NCHUNK // GRP


def _sc_mesh():
    return plsc.VectorSubcoreMesh(core_axis_name="c", subcore_axis_name="s")


def _deg_body(dst_hbm, out_hbm, dst_v, ones_v, zbuf, acc, sem):
    c = lax.axis_index("c")
    s = lax.axis_index("s")
    w = c * NS + s

    for r in range(ZR):
        for k in range(8):
            zbuf[r, pl.ds(k * 16, 16)] = jnp.zeros((16,), jnp.float32)

    for r in range(K):
        for k in range(8):
            ones_v[r, pl.ds(k * 16, 16)] = jnp.ones((16,), jnp.float32)

    def zcopy(i, _):
        pltpu.sync_copy(zbuf, acc.at[pl.ds(s * ROWS_PT + i * ZR, ZR)])
        return _
    lax.fori_loop(0, ROWS_PT // ZR, zcopy, None)
    plsc.subcore_barrier()

    pltpu.sync_copy(dst_hbm.at[w], dst_v)

    def dgrp(g, _):
        def issue(l, _2):
            pltpu.async_copy(ones_v, acc.at[dst_v.at[g, l]], sem, add=True)
            return _2
        lax.fori_loop(0, GRP, issue, None)

        def drain(l, _2):
            pltpu.make_async_copy(ones_v, acc.at[dst_v.at[g, 0]], sem).wait()
            return _2
        lax.fori_loop(0, GRP, drain, None)
        return _
    lax.fori_loop(0, NGRP, dgrp, None)
    plsc.subcore_barrier()

    pltpu.sync_copy(acc.at[pl.ds(s * ROWS_PT, ROWS_PT)],
                    out_hbm.at[c, pl.ds(s * ROWS_PT, ROWS_PT)])


def _deg_kernel(dst_r):
    return pl.kernel(
        _deg_body,
        out_type=jax.ShapeDtypeStruct((NC, N_PAD, H), jnp.float32),
        mesh=_sc_mesh(),
        scratch_types=[
            pltpu.VMEM((NGRP, GRP, K), jnp.int32),
            pltpu.VMEM((K, H), jnp.float32),
            pltpu.VMEM((ZR, H), jnp.float32),
            pltpu.VMEM_SHARED((N_PAD, H), jnp.float32),
            pltpu.SemaphoreType.DMA,
        ],
    )(dst_r)


def _scatter_kernel(m2, src_r, dst_r):
    def body(m2_hbm, src_hbm, dst_hbm, out_hbm,
             src_v, dst_v, rows0, rows1, rows2, rows3, zbuf, acc,
             gs0, gs1, gs2, gs3, ss0, ss1, ss2, ss3):
        c = lax.axis_index("c")
        s = lax.axis_index("s")
        w = c * NS + s
        rows = (rows0, rows1, rows2, rows3)
        gs = (gs0, gs1, gs2, gs3)
        ss = (ss0, ss1, ss2, ss3)

        for r in range(ZR):
            for k in range(8):
                zbuf[r, pl.ds(k * 16, 16)] = jnp.zeros((16,), jnp.float32)

        def zcopy(i, _):
            pltpu.sync_copy(zbuf, acc.at[pl.ds(s * ROWS_PT + i * ZR, ZR)])
            return _
        lax.fori_loop(0, ROWS_PT // ZR, zcopy, None)
        plsc.subcore_barrier()

        def gath(l, b):
            pltpu.async_copy(m2_hbm.at[src_v.at[l]], rows[b], gs[b])

        def gath_wait(b):
            pltpu.make_async_copy(m2_hbm.at[src_v.at[0]], rows[b],
                                  gs[b]).wait()

        def scat(l, b):
            pltpu.async_copy(rows[b], acc.at[dst_v.at[l]], ss[b], add=True)

        def scat_wait(b):
            pltpu.make_async_copy(rows[b], acc.at[dst_v.at[0]],
                                  ss[b]).wait()

        def group(g, _):
            pltpu.sync_copy(src_hbm.at[w, g], src_v)
            pltpu.sync_copy(dst_hbm.at[w, g], dst_v)
            gath(0, 0)
            gath(1, 1)
            gath(2, 2)
            gath_wait(0)
            scat(0, 0)
            gath(3, 3)
            gath_wait(1)
            scat(1, 1)

            def steps(i, _2):
                base = 4 * i + 2
                for d in range(4):
                    l = base + d
                    scat_wait(d)
                    gath(l + 2, d)
                    gath_wait((2 + d) % 4)
                    scat(l, (2 + d) % 4)
                return _2
            lax.fori_loop(0, (GRP - 4) // 4, steps, None)
            scat_wait(0)
            gath_wait((GRP - 2) % 4)
            scat(GRP - 2, (GRP - 2) % 4)
            scat_wait(1)
            gath_wait((GRP - 1) % 4)
            scat(GRP - 1, (GRP - 1) % 4)
            scat_wait(2)
            scat_wait(3)
            return _
        lax.fori_loop(0, NGRP, group, None)
        plsc.subcore_barrier()

        pltpu.sync_copy(acc.at[pl.ds(s * ROWS_PT, ROWS_PT)],
                        out_hbm.at[c, pl.ds(s * ROWS_PT, ROWS_PT)])

    return pl.kernel(
        body,
        out_type=jax.ShapeDtypeStruct((NC, N_PAD, H), jnp.float32),
        mesh=_sc_mesh(),
        scratch_types=[
            pltpu.VMEM((GRP, K), jnp.int32),
            pltpu.VMEM((GRP, K), jnp.int32),
            pltpu.VMEM((K, H), jnp.float32),
            pltpu.VMEM((K, H), jnp.float32),
            pltpu.VMEM((K, H), jnp.float32),
            pltpu.VMEM((K, H), jnp.float32),
            pltpu.VMEM((ZR, H), jnp.float32),
            pltpu.VMEM_SHARED((N_PAD, H), jnp.float32),
            pltpu.SemaphoreType.DMA,
            pltpu.SemaphoreType.DMA,
            pltpu.SemaphoreType.DMA,
            pltpu.SemaphoreType.DMA,
            pltpu.SemaphoreType.DMA,
            pltpu.SemaphoreType.DMA,
            pltpu.SemaphoreType.DMA,
            pltpu.SemaphoreType.DMA,
        ],
    )(m2, src_r, dst_r)


def _bn(h, g, b):
    mu = jnp.mean(h, axis=0, keepdims=True)
    var = jnp.mean((h - mu) ** 2, axis=0, keepdims=True)
    return (h - mu) * jax.lax.rsqrt(var + 1e-5) * g + b


def _tc_pro_body(x_ref, degp_ref, bng_ref, bnb_ref, w_ref,
                 dis_ref, m2_ref):
    dp = degp_ref[...]
    deg = dp[0, :N, 0:1] + dp[1, :N, 0:1] + 1.0
    dis = jax.lax.rsqrt(deg)
    dis_ref[...] = dis
    h_ = _bn(x_ref[...], bng_ref[...], bnb_ref[...])
    m2_ref[...] = dis * jnp.dot(h_, w_ref[...],
                                preferred_element_type=jnp.float32)


def _tc_mid_body(h_ref, m2_ref, parts_ref, dis_ref, b_ref,
                 bng_ref, bnb_ref, w_ref, hn_ref, m2n_ref):
    dis = dis_ref[...]
    pr = parts_ref[...]
    agg = pr[0, :N] + pr[1, :N] + m2_ref[...]
    out = dis * agg + b_ref[...]
    h = h_ref[...] + jnp.maximum(out, 0.0)
    hn_ref[...] = h
    h_ = _bn(h, bng_ref[...], bnb_ref[...])
    m2n_ref[...] = dis * jnp.dot(h_, w_ref[...],
                                 preferred_element_type=jnp.float32)


def _tc_epi_body(h_ref, m2_ref, parts_ref, dis_ref, b_ref, batch_ref,
                 bnfcg_ref, bnfcb_ref, wfc_ref, bfc_ref,
                 bnhg_ref, bnhb_ref, wcls_ref, bcls_ref, out_ref):
    dis = dis_ref[...]
    pr = parts_ref[...]
    agg = pr[0, :N] + pr[1, :N] + m2_ref[...]
    h = h_ref[...] + jnp.maximum(dis * agg + b_ref[...], 0.0)

    gid = jax.lax.broadcasted_iota(jnp.int32, (N, G), 1)
    p = (batch_ref[...] == gid).astype(jnp.float32)
    g = jax.lax.dot_general(p, h, (((0,), (0,)), ((), ())),
                            preferred_element_type=jnp.float32)

    g = _bn(g, bnfcg_ref[...], bnfcb_ref[...])
    g = jnp.maximum(jnp.dot(g, wfc_ref[...],
                            preferred_element_type=jnp.float32) + bfc_ref[...],
                    0.0)
    g = _bn(g, bnhg_ref[...], bnhb_ref[...])
    logits = jnp.dot(g, wcls_ref[...],
                     preferred_element_type=jnp.float32) + bcls_ref[...]
    m = jnp.max(logits, axis=-1, keepdims=True)
    lse = m + jnp.log(jnp.sum(jnp.exp(logits - m), axis=-1, keepdims=True))
    out_ref[...] = logits - lse


def _tc_call(body, out_shapes, *args):
    return pl.pallas_call(
        body,
        out_shape=out_shapes,
    )(*args)


def kernel(x, edge_index, batch, device, W0, b0, W1, b1, W2, b2,
           bn0_g, bn0_b, bn1_g, bn1_b, bn2_g, bn2_b,
           bnfc_g, bnfc_b, Wfc, bfc, bnh_g, bnh_b, Wcls, bcls):
    src_r = edge_index[0].astype(jnp.int32).reshape(NW, NGRP, GRP, K)
    dst_r = edge_index[1].astype(jnp.int32).reshape(NW, NGRP, GRP, K)
    batch2 = batch.astype(jnp.int32).reshape(N, 1)

    degp = _deg_kernel(dst_r)
    dis, m2 = _tc_call(
        _tc_pro_body,
        [jax.ShapeDtypeStruct((N, 1), jnp.float32),
         jax.ShapeDtypeStruct((N, H), jnp.float32)],
        x, degp, bn0_g.reshape(1, F), bn0_b.reshape(1, F), W0)

    h = x
    for bb, bg, bnb, W in ((b0, bn1_g, bn1_b, W1), (b1, bn2_g, bn2_b, W2)):
        parts = _scatter_kernel(m2, src_r, dst_r)
        h, m2 = _tc_call(
            _tc_mid_body,
            [jax.ShapeDtypeStruct((N, H), jnp.float32),
             jax.ShapeDtypeStruct((N, H), jnp.float32)],
            h, m2, parts, dis, bb.reshape(1, H),
            bg.reshape(1, H), bnb.reshape(1, H), W)

    parts = _scatter_kernel(m2, src_r, dst_r)
    out = _tc_call(
        _tc_epi_body,
        jax.ShapeDtypeStruct((G, C), jnp.float32),
        h, m2, parts, dis, b2.reshape(1, H), batch2,
        bnfc_g.reshape(1, H), bnfc_b.reshape(1, H), Wfc, bfc.reshape(1, H),
        bnh_g.reshape(1, H), bnh_b.reshape(1, H), Wcls, bcls.reshape(1, C))
    return out


# speedup vs baseline: 23.4242x; 1.0328x over previous
import functools

import jax
import jax.numpy as jnp
from jax import lax
from jax.experimental import pallas as pl
from jax.experimental.pallas import tpu as pltpu
from jax.experimental.pallas import tpu_sc as plsc

N = 10000
E = 320000
F = 128
H = 128
C = 10
G = 128

NC = 2
NS = 16
NW = NC * NS
EPT = E // NW
K = 50
NCHUNK = EPT // K
ROWS_PT = 632
N_PAD = NS * ROWS_PT
ZR = 8
GRP = 40
NGRP = NCHUNK // GRP


def _sc_mesh():
    return plsc.VectorSubcoreMesh(core_axis_name="c", subcore_axis_name="s")


def _deg_body(dst_hbm, out_hbm, dst_v, ones_v, zbuf, acc, sem):
    c = lax.axis_index("c")
    s = lax.axis_index("s")
    w = c * NS + s

    for r in range(ZR):
        for k in range(8):
            zbuf[r, pl.ds(k * 16, 16)] = jnp.zeros((16,), jnp.float32)

    for r in range(K):
        for k in range(8):
            ones_v[r, pl.ds(k * 16, 16)] = jnp.ones((16,), jnp.float32)

    def zissue(i, _):
        pltpu.async_copy(zbuf, acc.at[pl.ds(s * ROWS_PT + i * ZR, ZR)], sem)
        return _
    lax.fori_loop(0, ROWS_PT // ZR, zissue, None)

    def zdrain(i, _):
        pltpu.make_async_copy(zbuf, acc.at[pl.ds(s * ROWS_PT, ZR)],
                              sem).wait()
        return _
    lax.fori_loop(0, ROWS_PT // ZR, zdrain, None)
    plsc.subcore_barrier()

    pltpu.sync_copy(dst_hbm.at[w], dst_v)

    def dgrp(g, _):
        def issue(l, _2):
            pltpu.async_copy(ones_v, acc.at[dst_v.at[g, l]], sem, add=True)
            return _2
        lax.fori_loop(0, GRP, issue, None)

        def drain(l, _2):
            pltpu.make_async_copy(ones_v, acc.at[dst_v.at[g, 0]], sem).wait()
            return _2
        lax.fori_loop(0, GRP, drain, None)
        return _
    lax.fori_loop(0, NGRP, dgrp, None)
    plsc.subcore_barrier()

    pltpu.sync_copy(acc.at[pl.ds(s * ROWS_PT, ROWS_PT)],
                    out_hbm.at[c, pl.ds(s * ROWS_PT, ROWS_PT)])


def _deg_kernel(dst_r):
    return pl.kernel(
        _deg_body,
        out_type=jax.ShapeDtypeStruct((NC, N_PAD, H), jnp.float32),
        mesh=_sc_mesh(),
        scratch_types=[
            pltpu.VMEM((NGRP, GRP, K), jnp.int32),
            pltpu.VMEM((K, H), jnp.float32),
            pltpu.VMEM((ZR, H), jnp.float32),
            pltpu.VMEM_SHARED((N_PAD, H), jnp.float32),
            pltpu.SemaphoreType.DMA,
        ],
    )(dst_r)


def _scatter_kernel(m2, src_r, dst_r):
    def body(m2_hbm, src_hbm, dst_hbm, out_hbm,
             src_v, dst_v, rows0, rows1, rows2, rows3, zbuf, acc,
             gs0, gs1, gs2, gs3, ss0, ss1, ss2, ss3):
        c = lax.axis_index("c")
        s = lax.axis_index("s")
        w = c * NS + s
        rows = (rows0, rows1, rows2, rows3)
        gs = (gs0, gs1, gs2, gs3)
        ss = (ss0, ss1, ss2, ss3)

        for r in range(ZR):
            for k in range(8):
                zbuf[r, pl.ds(k * 16, 16)] = jnp.zeros((16,), jnp.float32)

        def zissue(i, _):
            pltpu.async_copy(zbuf, acc.at[pl.ds(s * ROWS_PT + i * ZR, ZR)],
                             gs0)
            return _
        lax.fori_loop(0, ROWS_PT // ZR, zissue, None)

        def zdrain(i, _):
            pltpu.make_async_copy(zbuf, acc.at[pl.ds(s * ROWS_PT, ZR)],
                                  gs0).wait()
            return _
        lax.fori_loop(0, ROWS_PT // ZR, zdrain, None)
        plsc.subcore_barrier()

        def gath(l, b):
            pltpu.async_copy(m2_hbm.at[src_v.at[l]], rows[b], gs[b])

        def gath_wait(b):
            pltpu.make_async_copy(m2_hbm.at[src_v.at[0]], rows[b],
                                  gs[b]).wait()

        def scat(l, b):
            pltpu.async_copy(rows[b], acc.at[dst_v.at[l]], ss[b], add=True)

        def scat_wait(b):
            pltpu.make_async_copy(rows[b], acc.at[dst_v.at[0]],
                                  ss[b]).wait()

        def group(g, _):
            pltpu.sync_copy(src_hbm.at[w, g], src_v)
            pltpu.sync_copy(dst_hbm.at[w, g], dst_v)
            gath(0, 0)
            gath(1, 1)
            gath(2, 2)
            gath_wait(0)
            scat(0, 0)
            gath(3, 3)
            gath_wait(1)
            scat(1, 1)

            def steps(i, _2):
                base = 4 * i + 2
                for d in range(4):
                    l = base + d
                    scat_wait(d)
                    gath(l + 2, d)
                    gath_wait((2 + d) % 4)
                    scat(l, (2 + d) % 4)
                return _2
            lax.fori_loop(0, (GRP - 4) // 4, steps, None)
            scat_wait(0)
            gath_wait((GRP - 2) % 4)
            scat(GRP - 2, (GRP - 2) % 4)
            scat_wait(1)
            gath_wait((GRP - 1) % 4)
            scat(GRP - 1, (GRP - 1) % 4)
            scat_wait(2)
            scat_wait(3)
            return _
        lax.fori_loop(0, NGRP, group, None)
        plsc.subcore_barrier()

        pltpu.sync_copy(acc.at[pl.ds(s * ROWS_PT, ROWS_PT)],
                        out_hbm.at[c, pl.ds(s * ROWS_PT, ROWS_PT)])

    return pl.kernel(
        body,
        out_type=jax.ShapeDtypeStruct((NC, N_PAD, H), jnp.float32),
        mesh=_sc_mesh(),
        scratch_types=[
            pltpu.VMEM((GRP, K), jnp.int32),
            pltpu.VMEM((GRP, K), jnp.int32),
            pltpu.VMEM((K, H), jnp.float32),
            pltpu.VMEM((K, H), jnp.float32),
            pltpu.VMEM((K, H), jnp.float32),
            pltpu.VMEM((K, H), jnp.float32),
            pltpu.VMEM((ZR, H), jnp.float32),
            pltpu.VMEM_SHARED((N_PAD, H), jnp.float32),
            pltpu.SemaphoreType.DMA,
            pltpu.SemaphoreType.DMA,
            pltpu.SemaphoreType.DMA,
            pltpu.SemaphoreType.DMA,
            pltpu.SemaphoreType.DMA,
            pltpu.SemaphoreType.DMA,
            pltpu.SemaphoreType.DMA,
            pltpu.SemaphoreType.DMA,
        ],
    )(m2, src_r, dst_r)


def _bn(h, g, b):
    mu = jnp.mean(h, axis=0, keepdims=True)
    var = jnp.mean((h - mu) ** 2, axis=0, keepdims=True)
    return (h - mu) * jax.lax.rsqrt(var + 1e-5) * g + b


def _tc_pro_body(x_ref, degp_ref, bng_ref, bnb_ref, w_ref,
                 dis_ref, m2_ref):
    dp = degp_ref[...]
    deg = dp[0, :N, 0:1] + dp[1, :N, 0:1] + 1.0
    dis = jax.lax.rsqrt(deg)
    dis_ref[...] = dis
    h_ = _bn(x_ref[...], bng_ref[...], bnb_ref[...])
    m2_ref[...] = dis * jnp.dot(h_, w_ref[...],
                                preferred_element_type=jnp.float32)


def _tc_mid_body(h_ref, m2_ref, parts_ref, dis_ref, b_ref,
                 bng_ref, bnb_ref, w_ref, hn_ref, m2n_ref):
    dis = dis_ref[...]
    pr = parts_ref[...]
    agg = pr[0, :N] + pr[1, :N] + m2_ref[...]
    out = dis * agg + b_ref[...]
    h = h_ref[...] + jnp.maximum(out, 0.0)
    hn_ref[...] = h
    h_ = _bn(h, bng_ref[...], bnb_ref[...])
    m2n_ref[...] = dis * jnp.dot(h_, w_ref[...],
                                 preferred_element_type=jnp.float32)


def _tc_epi_body(h_ref, m2_ref, parts_ref, dis_ref, b_ref, batch_ref,
                 bnfcg_ref, bnfcb_ref, wfc_ref, bfc_ref,
                 bnhg_ref, bnhb_ref, wcls_ref, bcls_ref, out_ref):
    dis = dis_ref[...]
    pr = parts_ref[...]
    agg = pr[0, :N] + pr[1, :N] + m2_ref[...]
    h = h_ref[...] + jnp.maximum(dis * agg + b_ref[...], 0.0)

    gid = jax.lax.broadcasted_iota(jnp.int32, (N, G), 1)
    p = (batch_ref[...] == gid).astype(jnp.float32)
    g = jax.lax.dot_general(p, h, (((0,), (0,)), ((), ())),
                            preferred_element_type=jnp.float32)

    g = _bn(g, bnfcg_ref[...], bnfcb_ref[...])
    g = jnp.maximum(jnp.dot(g, wfc_ref[...],
                            preferred_element_type=jnp.float32) + bfc_ref[...],
                    0.0)
    g = _bn(g, bnhg_ref[...], bnhb_ref[...])
    logits = jnp.dot(g, wcls_ref[...],
                     preferred_element_type=jnp.float32) + bcls_ref[...]
    m = jnp.max(logits, axis=-1, keepdims=True)
    lse = m + jnp.log(jnp.sum(jnp.exp(logits - m), axis=-1, keepdims=True))
    out_ref[...] = logits - lse


def _tc_call(body, out_shapes, *args):
    return pl.pallas_call(
        body,
        out_shape=out_shapes,
    )(*args)


def kernel(x, edge_index, batch, device, W0, b0, W1, b1, W2, b2,
           bn0_g, bn0_b, bn1_g, bn1_b, bn2_g, bn2_b,
           bnfc_g, bnfc_b, Wfc, bfc, bnh_g, bnh_b, Wcls, bcls):
    src_r = edge_index[0].astype(jnp.int32).reshape(NW, NGRP, GRP, K)
    dst_r = edge_index[1].astype(jnp.int32).reshape(NW, NGRP, GRP, K)
    batch2 = batch.astype(jnp.int32).reshape(N, 1)

    degp = _deg_kernel(dst_r)
    dis, m2 = _tc_call(
        _tc_pro_body,
        [jax.ShapeDtypeStruct((N, 1), jnp.float32),
         jax.ShapeDtypeStruct((N, H), jnp.float32)],
        x, degp, bn0_g.reshape(1, F), bn0_b.reshape(1, F), W0)

    h = x
    for bb, bg, bnb, W in ((b0, bn1_g, bn1_b, W1), (b1, bn2_g, bn2_b, W2)):
        parts = _scatter_kernel(m2, src_r, dst_r)
        h, m2 = _tc_call(
            _tc_mid_body,
            [jax.ShapeDtypeStruct((N, H), jnp.float32),
             jax.ShapeDtypeStruct((N, H), jnp.float32)],
            h, m2, parts, dis, bb.reshape(1, H),
            bg.reshape(1, H), bnb.reshape(1, H), W)

    parts = _scatter_kernel(m2, src_r, dst_r)
    out = _tc_call(
        _tc_epi_body,
        jax.ShapeDtypeStruct((G, C), jnp.float32),
        h, m2, parts, dis, b2.reshape(1, H), batch2,
        bnfc_g.reshape(1, H), bnfc_b.reshape(1, H), Wfc, bfc.reshape(1, H),
        bnh_g.reshape(1, H), bnh_b.reshape(1, H), Wcls, bcls.reshape(1, C))
    return out


# speedup vs baseline: 23.4858x; 1.0026x over previous
import functools

import jax
import jax.numpy as jnp
from jax import lax
from jax.experimental import pallas as pl
from jax.experimental.pallas import tpu as pltpu
from jax.experimental.pallas import tpu_sc as plsc

N = 10000
E = 320000
F = 128
H = 128
C = 10
G = 128

NC = 2
NS = 16
NW = NC * NS
EPT = E // NW
K = 50
NCHUNK = EPT // K
ROWS_PT = 632
N_PAD = NS * ROWS_PT
ZR = 8
GRP = 40
NGRP = NCHUNK // GRP


def _sc_mesh():
    return plsc.VectorSubcoreMesh(core_axis_name="c", subcore_axis_name="s")


def _deg_body(dst_hbm, out_hbm, dst_v, ones_v, zbuf, acc, sem):
    c = lax.axis_index("c")
    s = lax.axis_index("s")
    w = c * NS + s

    for r in range(ZR):
        for k in range(8):
            zbuf[r, pl.ds(k * 16, 16)] = jnp.zeros((16,), jnp.float32)

    for r in range(K):
        for k in range(8):
            ones_v[r, pl.ds(k * 16, 16)] = jnp.ones((16,), jnp.float32)

    def zissue(i, _):
        pltpu.async_copy(zbuf, acc.at[pl.ds(s * ROWS_PT + i * ZR, ZR)], sem)
        return _
    lax.fori_loop(0, ROWS_PT // ZR, zissue, None)

    def zdrain(i, _):
        pltpu.make_async_copy(zbuf, acc.at[pl.ds(s * ROWS_PT, ZR)],
                              sem).wait()
        return _
    lax.fori_loop(0, ROWS_PT // ZR, zdrain, None)
    plsc.subcore_barrier()

    pltpu.sync_copy(dst_hbm.at[w], dst_v)

    def dgrp(g, _):
        def issue(l, _2):
            pltpu.async_copy(ones_v, acc.at[dst_v.at[g, l]], sem, add=True)
            return _2
        lax.fori_loop(0, GRP, issue, None)

        def drain(l, _2):
            pltpu.make_async_copy(ones_v, acc.at[dst_v.at[g, 0]], sem).wait()
            return _2
        lax.fori_loop(0, GRP, drain, None)
        return _
    lax.fori_loop(0, NGRP, dgrp, None)
    plsc.subcore_barrier()

    pltpu.sync_copy(acc.at[pl.ds(s * ROWS_PT, ROWS_PT)],
                    out_hbm.at[c, pl.ds(s * ROWS_PT, ROWS_PT)])


def _deg_kernel(dst_r):
    return pl.kernel(
        _deg_body,
        out_type=jax.ShapeDtypeStruct((NC, N_PAD, H), jnp.float32),
        mesh=_sc_mesh(),
        scratch_types=[
            pltpu.VMEM((NGRP, GRP, K), jnp.int32),
            pltpu.VMEM((K, H), jnp.float32),
            pltpu.VMEM((ZR, H), jnp.float32),
            pltpu.VMEM_SHARED((N_PAD, H), jnp.float32),
            pltpu.SemaphoreType.DMA,
        ],
    )(dst_r)


def _scatter_kernel(m2, src_r, dst_r):
    def body(m2_hbm, src_hbm, dst_hbm, out_hbm,
             src_v, dst_v, rows0, rows1, rows2, rows3, zbuf, acc,
             gs0, gs1, gs2, gs3, ss0, ss1, ss2, ss3):
        c = lax.axis_index("c")
        s = lax.axis_index("s")
        w = c * NS + s
        rows = (rows0, rows1, rows2, rows3)
        gs = (gs0, gs1, gs2, gs3)
        ss = (ss0, ss1, ss2, ss3)

        for r in range(ZR):
            for k in range(8):
                zbuf[r, pl.ds(k * 16, 16)] = jnp.zeros((16,), jnp.float32)

        def zissue(i, _):
            pltpu.async_copy(zbuf, acc.at[pl.ds(s * ROWS_PT + i * ZR, ZR)],
                             gs0)
            return _
        lax.fori_loop(0, ROWS_PT // ZR, zissue, None)

        def zdrain(i, _):
            pltpu.make_async_copy(zbuf, acc.at[pl.ds(s * ROWS_PT, ZR)],
                                  gs0).wait()
            return _
        lax.fori_loop(0, ROWS_PT // ZR, zdrain, None)
        plsc.subcore_barrier()

        def gath(l, b):
            pltpu.async_copy(m2_hbm.at[src_v.at[l]], rows[b], gs[b])

        def gath_wait(b):
            pltpu.make_async_copy(m2_hbm.at[src_v.at[0]], rows[b],
                                  gs[b]).wait()

        def scat(l, b):
            pltpu.async_copy(rows[b], acc.at[dst_v.at[l]], ss[b], add=True)

        def scat_wait(b):
            pltpu.make_async_copy(rows[b], acc.at[dst_v.at[0]],
                                  ss[b]).wait()

        def group(g, _):
            pltpu.sync_copy(src_hbm.at[w, g], src_v)
            pltpu.sync_copy(dst_hbm.at[w, g], dst_v)
            gath(0, 0)
            gath(1, 1)
            gath(2, 2)
            gath_wait(0)
            scat(0, 0)
            gath(3, 3)
            gath_wait(1)
            scat(1, 1)

            def steps(i, _2):
                base = 4 * i + 2
                for d in range(4):
                    l = base + d
                    scat_wait(d)
                    gath(l + 2, d)
                    gath_wait((2 + d) % 4)
                    scat(l, (2 + d) % 4)
                return _2
            lax.fori_loop(0, (GRP - 4) // 4, steps, None)
            scat_wait(0)
            gath_wait((GRP - 2) % 4)
            scat(GRP - 2, (GRP - 2) % 4)
            scat_wait(1)
            gath_wait((GRP - 1) % 4)
            scat(GRP - 1, (GRP - 1) % 4)
            scat_wait(2)
            scat_wait(3)
            return _
        lax.fori_loop(0, NGRP, group, None)
        plsc.subcore_barrier()

        pltpu.sync_copy(acc.at[pl.ds(s * ROWS_PT, ROWS_PT)],
                        out_hbm.at[c, pl.ds(s * ROWS_PT, ROWS_PT)])

    return pl.kernel(
        body,
        out_type=jax.ShapeDtypeStruct((NC, N_PAD, H), jnp.float32),
        mesh=_sc_mesh(),
        scratch_types=[
            pltpu.VMEM((GRP, K), jnp.int32),
            pltpu.VMEM((GRP, K), jnp.int32),
            pltpu.VMEM((K, H), jnp.float32),
            pltpu.VMEM((K, H), jnp.float32),
            pltpu.VMEM((K, H), jnp.float32),
            pltpu.VMEM((K, H), jnp.float32),
            pltpu.VMEM((ZR, H), jnp.float32),
            pltpu.VMEM_SHARED((N_PAD, H), jnp.float32),
            pltpu.SemaphoreType.DMA,
            pltpu.SemaphoreType.DMA,
            pltpu.SemaphoreType.DMA,
            pltpu.SemaphoreType.DMA,
            pltpu.SemaphoreType.DMA,
            pltpu.SemaphoreType.DMA,
            pltpu.SemaphoreType.DMA,
            pltpu.SemaphoreType.DMA,
        ],
    )(m2, src_r, dst_r)


def _bn(h, g, b):
    mu = jnp.mean(h, axis=0, keepdims=True)
    var = jnp.mean((h - mu) ** 2, axis=0, keepdims=True)
    return (h - mu) * jax.lax.rsqrt(var + 1e-5) * g + b


def _tc_pre_body(x_ref, bng_ref, bnb_ref, w_ref, mraw_ref):
    h_ = _bn(x_ref[...], bng_ref[...], bnb_ref[...])
    mraw_ref[...] = jnp.dot(h_, w_ref[...],
                            preferred_element_type=jnp.float32)


def _tc_scale_body(degp_ref, mraw_ref, dis_ref, m2_ref):
    dp = degp_ref[...]
    deg = dp[0, :N, 0:1] + dp[1, :N, 0:1] + 1.0
    dis = jax.lax.rsqrt(deg)
    dis_ref[...] = dis
    m2_ref[...] = dis * mraw_ref[...]


def _tc_mid_body(h_ref, m2_ref, parts_ref, dis_ref, b_ref,
                 bng_ref, bnb_ref, w_ref, hn_ref, m2n_ref):
    dis = dis_ref[...]
    pr = parts_ref[...]
    agg = pr[0, :N] + pr[1, :N] + m2_ref[...]
    out = dis * agg + b_ref[...]
    h = h_ref[...] + jnp.maximum(out, 0.0)
    hn_ref[...] = h
    h_ = _bn(h, bng_ref[...], bnb_ref[...])
    m2n_ref[...] = dis * jnp.dot(h_, w_ref[...],
                                 preferred_element_type=jnp.float32)


def _tc_epi_body(h_ref, m2_ref, parts_ref, dis_ref, b_ref, batch_ref,
                 bnfcg_ref, bnfcb_ref, wfc_ref, bfc_ref,
                 bnhg_ref, bnhb_ref, wcls_ref, bcls_ref, out_ref):
    dis = dis_ref[...]
    pr = parts_ref[...]
    agg = pr[0, :N] + pr[1, :N] + m2_ref[...]
    h = h_ref[...] + jnp.maximum(dis * agg + b_ref[...], 0.0)

    gid = jax.lax.broadcasted_iota(jnp.int32, (N, G), 1)
    p = (batch_ref[...] == gid).astype(jnp.float32)
    g = jax.lax.dot_general(p, h, (((0,), (0,)), ((), ())),
                            preferred_element_type=jnp.float32)

    g = _bn(g, bnfcg_ref[...], bnfcb_ref[...])
    g = jnp.maximum(jnp.dot(g, wfc_ref[...],
                            preferred_element_type=jnp.float32) + bfc_ref[...],
                    0.0)
    g = _bn(g, bnhg_ref[...], bnhb_ref[...])
    logits = jnp.dot(g, wcls_ref[...],
                     preferred_element_type=jnp.float32) + bcls_ref[...]
    m = jnp.max(logits, axis=-1, keepdims=True)
    lse = m + jnp.log(jnp.sum(jnp.exp(logits - m), axis=-1, keepdims=True))
    out_ref[...] = logits - lse


def _tc_call(body, out_shapes, *args):
    return pl.pallas_call(
        body,
        out_shape=out_shapes,
    )(*args)


def kernel(x, edge_index, batch, device, W0, b0, W1, b1, W2, b2,
           bn0_g, bn0_b, bn1_g, bn1_b, bn2_g, bn2_b,
           bnfc_g, bnfc_b, Wfc, bfc, bnh_g, bnh_b, Wcls, bcls):
    src_r = edge_index[0].astype(jnp.int32).reshape(NW, NGRP, GRP, K)
    dst_r = edge_index[1].astype(jnp.int32).reshape(NW, NGRP, GRP, K)
    batch2 = batch.astype(jnp.int32).reshape(N, 1)

    degp = _deg_kernel(dst_r)
    m_raw = _tc_call(
        _tc_pre_body,
        jax.ShapeDtypeStruct((N, H), jnp.float32),
        x, bn0_g.reshape(1, F), bn0_b.reshape(1, F), W0)
    dis, m2 = _tc_call(
        _tc_scale_body,
        [jax.ShapeDtypeStruct((N, 1), jnp.float32),
         jax.ShapeDtypeStruct((N, H), jnp.float32)],
        degp, m_raw)

    h = x
    for bb, bg, bnb, W in ((b0, bn1_g, bn1_b, W1), (b1, bn2_g, bn2_b, W2)):
        parts = _scatter_kernel(m2, src_r, dst_r)
        h, m2 = _tc_call(
            _tc_mid_body,
            [jax.ShapeDtypeStruct((N, H), jnp.float32),
             jax.ShapeDtypeStruct((N, H), jnp.float32)],
            h, m2, parts, dis, bb.reshape(1, H),
            bg.reshape(1, H), bnb.reshape(1, H), W)

    parts = _scatter_kernel(m2, src_r, dst_r)
    out = _tc_call(
        _tc_epi_body,
        jax.ShapeDtypeStruct((G, C), jnp.float32),
        h, m2, parts, dis, b2.reshape(1, H), batch2,
        bnfc_g.reshape(1, H), bnfc_b.reshape(1, H), Wfc, bfc.reshape(1, H),
        bnh_g.reshape(1, H), bnh_b.reshape(1, H), Wcls, bcls.reshape(1, C))
    return out
